# baseline (speedup 1.0000x reference)
import jax
import jax.numpy as jnp
from jax import lax
from jax.experimental import pallas as pl
from jax.experimental.pallas import tpu as pltpu
from jax.experimental.pallas import tpu_sc as plsc

N_ROWS = 100000
N_COLS = 128
NUM_SEGS = 512

NC = 2
NS = 16
NW = NC * NS
L = 16

TC_BLK = 1024
TC_NBLK = 49
S_TC = TC_BLK * TC_NBLK

SC_BASE = S_TC
N_SC = N_ROWS - S_TC
COLS_PER_CORE = N_COLS // NC
NJ = COLS_PER_CORE // L
CHUNK = 640
FULL_CHUNKS = N_SC // CHUNK
REM_ROWS = N_SC - FULL_CHUNKS * CHUNK
REM_TILE = FULL_CHUNKS % NS
CHUNKS_PER_TILE = (FULL_CHUNKS + NS - 1) // NS
ACC_COLS = COLS_PER_CORE + L
SEGS_PER_TILE = NUM_SEGS // NS
SEGS_PER_W = NUM_SEGS // NW


def _tc_body(x_ref, ids_ref, sum_ref, cnt_ref):
    i = pl.program_id(0)
    ids = ids_ref[0, 0, :]
    seg_iota = lax.broadcasted_iota(jnp.int32, (NUM_SEGS, TC_BLK), 0)
    oh = (seg_iota == ids[None, :]).astype(jnp.float32)

    @pl.when(i == 0)
    def _():
        sum_ref[...] = jnp.zeros_like(sum_ref)
        cnt_ref[...] = jnp.zeros_like(cnt_ref)

    sum_ref[...] += jnp.dot(oh, x_ref[...],
                            preferred_element_type=jnp.float32)
    cnt_ref[0:1, :] += jnp.sum(oh, axis=1)[None, :]


def _stage1_body(x_hbm, batch_hbm, part_hbm,
                 acc, rows0, rows1, idx0, idx1, idmap, zbuf,
                 shared, sem0, sem1):
    cid = lax.axis_index("c")
    tid = lax.axis_index("s")
    col0 = cid * COLS_PER_CORE

    zero = jnp.zeros((L,), jnp.float32)
    cnt_one = jnp.full((L,), 1.0, jnp.float32)
    cnt_grp = jnp.full((L,), float(L), jnp.float32)
    cnt_blk = jnp.full((L,), float(2 * L), jnp.float32)

    def zero_body(s, _):
        for j in range(ACC_COLS // L):
            acc[s, pl.ds(j * L, L)] = zero
        return 0

    lax.fori_loop(0, NUM_SEGS, zero_body, 0)

    def accum_rows(rows, idx, nrows):
        def sum_run(r0, n, s_first, cnt_vec):
            accs = [rows[r0, pl.ds(j * L, L)] for j in range(NJ)]
            for k in range(1, n):
                for j in range(NJ):
                    accs[j] = accs[j] + rows[r0 + k, pl.ds(j * L, L)]
            for j in range(NJ):
                plsc.addupdate(acc.at[s_first, pl.ds(j * L, L)], accs[j])
            plsc.addupdate(acc.at[s_first, pl.ds(COLS_PER_CORE, L)], cnt_vec)

        def group16(r0, segvec):
            s_first = segvec[0]
            s_last = segvec[L - 1]

            @pl.when(s_first == s_last)
            def _():
                sum_run(r0, L, s_first, cnt_grp)

            @pl.when(s_first != s_last)
            def _():
                for k in range(L):
                    seg = segvec[k]
                    vals = [rows[r0 + k, pl.ds(j * L, L)] for j in range(NJ)]
                    for j in range(NJ):
                        plsc.addupdate(acc.at[seg, pl.ds(j * L, L)], vals[j])
                    plsc.addupdate(acc.at[seg, pl.ds(COLS_PER_CORE, L)],
                                   cnt_one)

        def block_body(blk, _):
            r0 = blk * (2 * L)
            seg_a = idx[pl.ds(r0, L)]
            seg_b = idx[pl.ds(r0 + L, L)]
            s_first = seg_a[0]
            s_last = seg_b[L - 1]

            @pl.when(s_first == s_last)
            def _():
                sum_run(r0, 2 * L, s_first, cnt_blk)

            @pl.when(s_first != s_last)
            def _():
                group16(r0, seg_a)
                group16(r0 + L, seg_b)

            return 0

        lax.fori_loop(0, nrows // (2 * L), block_body, 0)

    bufs = ((rows0, idx0, sem0), (rows1, idx1, sem1))

    def copies(i, b):
        base = SC_BASE + (tid + i * NS) * CHUNK
        rows_b, idx_b, sem_b = bufs[b]
        return (
            pltpu.make_async_copy(batch_hbm.at[pl.ds(base, CHUNK)],
                                  idx_b, sem_b),
            pltpu.make_async_copy(
                x_hbm.at[pl.ds(base, CHUNK), pl.ds(col0, COLS_PER_CORE)],
                rows_b, sem_b),
        )

    def start_copies(i, b):
        @pl.when(tid + i * NS < FULL_CHUNKS)
        def _():
            for cp in copies(i, b):
                cp.start()

    for b in range(2):
        start_copies(b, b)

    def outer(io, _):
        for b in range(2):
            i = io * 2 + b

            @pl.when(tid + i * NS < FULL_CHUNKS)
            def _():
                for cp in copies(i, b):
                    cp.wait()
                accum_rows(bufs[b][0], bufs[b][1], CHUNK)
                start_copies(i + 2, b)

        return 0

    lax.fori_loop(0, (CHUNKS_PER_TILE + 1) // 2, outer, 0)

    @pl.when(tid == REM_TILE)
    def _():
        base = SC_BASE + FULL_CHUNKS * CHUNK
        pltpu.sync_copy(batch_hbm.at[pl.ds(base, REM_ROWS)],
                        idx0.at[pl.ds(0, REM_ROWS)])
        pltpu.sync_copy(
            x_hbm.at[pl.ds(base, REM_ROWS), pl.ds(col0, COLS_PER_CORE)],
            rows0.at[pl.ds(0, REM_ROWS)])
        accum_rows(rows0, idx0, REM_ROWS)

    seg0 = tid * SEGS_PER_TILE

    def fill_idmap(p, _):
        iota = lax.iota(jnp.int32, L)

        def fill16(g, _):
            idmap[p, pl.ds(g * L, L)] = iota + (p * 128 + g * L)
            return 0

        lax.fori_loop(0, 128 // L, fill16, 0)
        return 0

    lax.fori_loop(0, NUM_SEGS // 128, fill_idmap, 0)

    def zero_zbuf(s, _):
        for j in range(ACC_COLS // L):
            zbuf[s, pl.ds(j * L, L)] = zero
        return 0

    lax.fori_loop(0, SEGS_PER_TILE, zero_zbuf, 0)

    pltpu.sync_copy(zbuf, shared.at[pl.ds(seg0, SEGS_PER_TILE)])
    plsc.subcore_barrier()

    for p in range(NUM_SEGS // 128):
        pltpu.sync_copy(acc.at[pl.ds(p * 128, 128)],
                        shared.at[idmap.at[p]], add=True)
    plsc.subcore_barrier()

    pltpu.sync_copy(shared.at[pl.ds(seg0, SEGS_PER_TILE)],
                    part_hbm.at[cid, pl.ds(seg0, SEGS_PER_TILE)])


def _stage2_body(part_hbm, tcsum_hbm, tccnt_hbm, out_hbm,
                 p0, p1, tcs, tcc, obuf):
    cid = lax.axis_index("c")
    tid = lax.axis_index("s")
    wid = cid * NS + tid
    seg0 = wid * SEGS_PER_W

    pltpu.sync_copy(part_hbm.at[0, pl.ds(seg0, SEGS_PER_W)], p0)
    pltpu.sync_copy(part_hbm.at[1, pl.ds(seg0, SEGS_PER_W)], p1)
    pltpu.sync_copy(tcsum_hbm.at[pl.ds(seg0, SEGS_PER_W)], tcs)
    pltpu.sync_copy(tccnt_hbm.at[pl.ds(0, 1), pl.ds(seg0, SEGS_PER_W)], tcc)

    tcv = tcc[0, pl.ds(0, L)]
    for s in range(SEGS_PER_W):
        cnt = p0[s, pl.ds(COLS_PER_CORE, L)] + jnp.broadcast_to(tcv[s], (L,))
        recip = jnp.float32(1.0) / jnp.maximum(cnt, jnp.float32(1.0))
        for j in range(NJ):
            obuf[s, pl.ds(j * L, L)] = (
                p0[s, pl.ds(j * L, L)] + tcs[s, pl.ds(j * L, L)]) * recip
        for j in range(NJ):
            obuf[s, pl.ds(COLS_PER_CORE + j * L, L)] = (
                p1[s, pl.ds(j * L, L)]
                + tcs[s, pl.ds(COLS_PER_CORE + j * L, L)]) * recip

    pltpu.sync_copy(obuf, out_hbm.at[pl.ds(seg0, SEGS_PER_W)])


@jax.jit
def _pool(x, batch):
    ids3 = batch[:S_TC].reshape(TC_NBLK, 1, TC_BLK)
    tcsum, tccnt = pl.pallas_call(
        _tc_body,
        grid=(TC_NBLK,),
        in_specs=[
            pl.BlockSpec((TC_BLK, N_COLS), lambda i: (i, 0)),
            pl.BlockSpec((1, 1, TC_BLK), lambda i: (i, 0, 0)),
        ],
        out_specs=[
            pl.BlockSpec((NUM_SEGS, N_COLS), lambda i: (0, 0)),
            pl.BlockSpec((8, NUM_SEGS), lambda i: (0, 0)),
        ],
        out_shape=[
            jax.ShapeDtypeStruct((NUM_SEGS, N_COLS), jnp.float32),
            jax.ShapeDtypeStruct((8, NUM_SEGS), jnp.float32),
        ],
        compiler_params=pltpu.CompilerParams(
            dimension_semantics=("arbitrary",)),
    )(x, ids3)

    mesh = plsc.VectorSubcoreMesh(core_axis_name="c", subcore_axis_name="s",
                                  num_cores=NC, num_subcores=NS)
    partials = pl.kernel(
        _stage1_body,
        out_type=jax.ShapeDtypeStruct((NC, NUM_SEGS, ACC_COLS), jnp.float32),
        mesh=mesh,
        scratch_types=[
            pltpu.VMEM((NUM_SEGS, ACC_COLS), jnp.float32),
            pltpu.VMEM((CHUNK, COLS_PER_CORE), jnp.float32),
            pltpu.VMEM((CHUNK, COLS_PER_CORE), jnp.float32),
            pltpu.VMEM((CHUNK,), jnp.int32),
            pltpu.VMEM((CHUNK,), jnp.int32),
            pltpu.VMEM((NUM_SEGS // 128, 128), jnp.int32),
            pltpu.VMEM((SEGS_PER_TILE, ACC_COLS), jnp.float32),
            pltpu.VMEM_SHARED((NUM_SEGS, ACC_COLS), jnp.float32),
            pltpu.SemaphoreType.DMA,
            pltpu.SemaphoreType.DMA,
        ],
        compiler_params=pltpu.CompilerParams(use_tc_tiling_on_sc=False),
    )(x, batch)

    return pl.kernel(
        _stage2_body,
        out_type=jax.ShapeDtypeStruct((NUM_SEGS, N_COLS), jnp.float32),
        mesh=mesh,
        scratch_types=[
            pltpu.VMEM((SEGS_PER_W, ACC_COLS), jnp.float32),
            pltpu.VMEM((SEGS_PER_W, ACC_COLS), jnp.float32),
            pltpu.VMEM((SEGS_PER_W, N_COLS), jnp.float32),
            pltpu.VMEM((1, SEGS_PER_W), jnp.float32),
            pltpu.VMEM((SEGS_PER_W, N_COLS), jnp.float32),
        ],
        compiler_params=pltpu.CompilerParams(use_tc_tiling_on_sc=False),
    )(partials, tcsum, tccnt)


def kernel(x, batch):
    return _pool(x, batch.astype(jnp.int32))
